# Initial kernel scaffold; baseline (speedup 1.0000x reference)
import jax
import jax.numpy as jnp
from jax.experimental import pallas as pl


def kernel(ego_states, x, edge_index, depths, batch, params):
    raise NotImplementedError("write your pallas kernel here")


# speedup vs baseline: 25.2121x; 25.2121x over previous
import functools

import jax
import jax.numpy as jnp
from jax import lax
from jax.experimental import pallas as pl
from jax.experimental.pallas import tpu as pltpu
from jax.experimental.pallas import tpu_sc as plsc

F32 = jnp.float32

_NC = 2
_NS = 16
_CH = 80


def _dot_t(x, w):
    return lax.dot_general(x, w, (((1,), (1,)), ((), ())),
                           preferred_element_type=F32)


def _pre_body(x, wi0, wo0, wu0, wf1, uf1, wi1, wo1, wu1,
              bi0, bo0, bu0, bf1, buf1, bi1, bo1, bu1,
              h1_o, s_o, a_o, xwi_o, xwo_o, xwu_o):
    xb = x[...]
    i1 = jnp.maximum(_dot_t(xb, wi0[...]) + bi0[...], 0.0)
    o1 = jnp.maximum(_dot_t(xb, wo0[...]) + bo0[...], 0.0)
    u1 = jnp.tanh(_dot_t(xb, wu0[...]) + bu0[...])
    c1 = i1 * u1
    h1 = o1 * jnp.tanh(c1)
    h1_o[...] = h1
    s_o[...] = jnp.concatenate(
        [_dot_t(h1, uf1[...]) + buf1[...], c1], axis=1)
    a_o[...] = _dot_t(xb, wf1[...]) + bf1[...]
    xwi_o[...] = _dot_t(xb, wi1[...]) + bi1[...]
    xwo_o[...] = _dot_t(xb, wo1[...]) + bo1[...]
    xwu_o[...] = _dot_t(xb, wu1[...]) + bu1[...]


def _pre_call(x, mats, vecs, bn, interpret=False):
    n, d = x.shape
    h = mats[0].shape[0]
    nb = n // bn
    row_spec = pl.BlockSpec((bn, h), lambda i: (i, 0))
    mat_spec = pl.BlockSpec((h, d), lambda i: (0, 0))
    vec_spec = pl.BlockSpec((1, h), lambda i: (0, 0))
    return pl.pallas_call(
        _pre_body,
        grid=(nb,),
        in_specs=[pl.BlockSpec((bn, d), lambda i: (i, 0))]
        + [mat_spec] * 8 + [vec_spec] * 8,
        out_specs=[row_spec,
                   pl.BlockSpec((bn, 2 * h), lambda i: (i, 0)),
                   row_spec, row_spec, row_spec, row_spec],
        out_shape=[jax.ShapeDtypeStruct((n, h), F32),
                   jax.ShapeDtypeStruct((n, 2 * h), F32),
                   jax.ShapeDtypeStruct((n, h), F32),
                   jax.ShapeDtypeStruct((n, h), F32),
                   jax.ShapeDtypeStruct((n, h), F32),
                   jax.ShapeDtypeStruct((n, h), F32)],
        interpret=interpret,
    )(x, *mats, *vecs)


def _sc_common(n, h, acc, zbuf, sid, zb, nrt):
    def zrow(r, carry):
        for j in range(h // 16):
            zbuf[r, pl.ds(j * 16, 16)] = jnp.zeros((16,), F32)
        return carry
    lax.fori_loop(0, zb, zrow, 0)
    r0 = sid * nrt
    for k in range(nrt // zb):
        pltpu.sync_copy(zbuf, acc.at[pl.ds(r0 + k * zb, zb)])
    return r0


def _sc_dump(acc, zbuf, out_hbm, cid, r0, zb, nrt):
    for k in range(nrt // zb):
        rr = r0 + k * zb
        pltpu.sync_copy(acc.at[pl.ds(rr, zb)], zbuf)
        pltpu.sync_copy(zbuf, out_hbm.at[cid, pl.ds(rr, zb)])


def _make_ht_call(np_pad, h, ep):
    nw = _NC * _NS
    epw = ep // nw
    nch = epw // _CH
    nrt = np_pad // _NS
    zb = 128
    mesh = plsc.VectorSubcoreMesh(core_axis_name="c", subcore_axis_name="s")

    @functools.partial(
        pl.kernel,
        out_type=jax.ShapeDtypeStruct((_NC, np_pad, h), F32),
        mesh=mesh,
        scratch_types=[
            pltpu.VMEM((_CH,), jnp.int32),
            pltpu.VMEM((_CH,), jnp.int32),
            pltpu.VMEM((_CH, h), F32),
            pltpu.VMEM((zb, h), F32),
            pltpu.VMEM_SHARED((np_pad + 8, h), F32),
            pltpu.SemaphoreType.DMA,
        ],
    )
    def ht_k(src_hbm, dst_hbm, tab_hbm, out_hbm,
             idx_s, idx_d, rows, zbuf, acc, sem):
        cid = lax.axis_index("c")
        sid = lax.axis_index("s")
        r0 = _sc_common(np_pad, h, acc, zbuf, sid, zb, nrt)
        plsc.subcore_barrier()
        base = (cid * _NS + sid) * epw

        def chunk(i, carry):
            off = base + i * _CH
            pltpu.sync_copy(src_hbm.at[pl.ds(off, _CH)], idx_s)
            pltpu.sync_copy(dst_hbm.at[pl.ds(off, _CH)], idx_d)
            pltpu.async_copy(tab_hbm.at[idx_s], rows, sem).wait()
            pltpu.sync_copy(rows, acc.at[idx_d], add=True)
            return carry
        lax.fori_loop(0, nch, chunk, 0)
        plsc.subcore_barrier()
        _sc_dump(acc, zbuf, out_hbm, cid, r0, zb, nrt)

    return ht_k


def _make_msg_call(np_pad, h, ep):
    nw = _NC * _NS
    epw = ep // nw
    nch = epw // _CH
    nrt = np_pad // _NS
    zb = 128
    mesh = plsc.VectorSubcoreMesh(core_axis_name="c", subcore_axis_name="s")

    @functools.partial(
        pl.kernel,
        out_type=jax.ShapeDtypeStruct((_NC, np_pad, h), F32),
        mesh=mesh,
        scratch_types=[
            pltpu.VMEM((_CH,), jnp.int32),
            pltpu.VMEM((_CH,), jnp.int32),
            pltpu.VMEM((_CH, 2 * h), F32),
            pltpu.VMEM((_CH, h), F32),
            pltpu.VMEM((zb, h), F32),
            pltpu.VMEM_SHARED((np_pad + 8, h), F32),
            pltpu.SemaphoreType.DMA,
            pltpu.SemaphoreType.DMA,
        ],
    )
    def msg_k(src_hbm, dst_hbm, s_hbm, a_hbm, out_hbm,
              idx_s, idx_d, srows, arows, zbuf, acc, sem1, sem2):
        cid = lax.axis_index("c")
        sid = lax.axis_index("s")
        r0 = _sc_common(np_pad, h, acc, zbuf, sid, zb, nrt)
        plsc.subcore_barrier()
        base = (cid * _NS + sid) * epw

        def chunk(i, carry):
            off = base + i * _CH
            pltpu.sync_copy(src_hbm.at[pl.ds(off, _CH)], idx_s)
            pltpu.sync_copy(dst_hbm.at[pl.ds(off, _CH)], idx_d)
            cp1 = pltpu.async_copy(s_hbm.at[idx_s], srows, sem1)
            cp2 = pltpu.async_copy(a_hbm.at[idx_d], arows, sem2)
            cp1.wait()
            cp2.wait()

            def erow(e, c2):
                for j in range(h // 16):
                    sl = pl.ds(j * 16, 16)
                    a = arows[e, sl]
                    b = srows[e, sl]
                    cc = srows[e, pl.ds(h + j * 16, 16)]
                    arows[e, sl] = jnp.maximum(a + b, 0.0) * cc
                return c2
            lax.fori_loop(0, _CH, erow, 0)
            pltpu.sync_copy(arows, acc.at[idx_d], add=True)
            return carry
        lax.fori_loop(0, nch, chunk, 0)
        plsc.subcore_barrier()
        _sc_dump(acc, zbuf, out_hbm, cid, r0, zb, nrt)

    return msg_k


def _post_body(xwi, xwo, xwu, htp, msgp, batchr, ui1, uo1, uu1,
               pooled_o, counts_o):
    step = pl.program_id(0)
    ht = htp[0] + htp[1]
    i2 = jnp.maximum(xwi[...] + _dot_t(ht, ui1[...]), 0.0)
    o2 = jnp.maximum(xwo[...] + _dot_t(ht, uo1[...]), 0.0)
    u2 = jnp.tanh(xwu[...] + _dot_t(ht, uu1[...]))
    c2 = i2 * u2 + msgp[0] + msgp[1]
    h2 = o2 * jnp.tanh(c2)
    g = pooled_o.shape[0]
    bn = h2.shape[0]
    b = batchr[0]
    gids = lax.broadcasted_iota(jnp.int32, (g, bn), 0)
    onehot = (b == gids).astype(F32)
    acc = lax.dot_general(onehot, h2, (((1,), (0,)), ((), ())),
                          preferred_element_type=F32)
    cnt = jnp.sum(onehot, axis=1)[None, :]

    @pl.when(step == 0)
    def _init():
        pooled_o[...] = jnp.zeros_like(pooled_o)
        counts_o[...] = jnp.zeros_like(counts_o)

    pooled_o[...] += acc
    counts_o[...] += cnt


def _post_call(xwi, xwo, xwu, htp, msgp, batch3, mats, g, bn,
               interpret=False):
    n, h = xwi.shape
    nb = n // bn
    row_spec = pl.BlockSpec((bn, h), lambda i: (i, 0))
    part_spec = pl.BlockSpec((_NC, bn, h), lambda i: (0, i, 0))
    mat_spec = pl.BlockSpec((h, h), lambda i: (0, 0))
    return pl.pallas_call(
        _post_body,
        grid=(nb,),
        in_specs=[row_spec, row_spec, row_spec, part_spec, part_spec,
                  pl.BlockSpec((1, 1, bn), lambda i: (i, 0, 0)),
                  mat_spec, mat_spec, mat_spec],
        out_specs=[pl.BlockSpec((g, h), lambda i: (0, 0)),
                   pl.BlockSpec((1, g), lambda i: (0, 0))],
        out_shape=[jax.ShapeDtypeStruct((g, h), F32),
                   jax.ShapeDtypeStruct((1, g), F32)],
        interpret=interpret,
    )(xwi, xwo, xwu, htp, msgp, batch3, *mats)


def _final_body(pooled, counts, whp, bhp, ego, w1, b1, w2, b2, w3, b3,
                out_o):
    cnt = jnp.maximum(counts[...], 1.0)
    hg = pooled[...] / cnt.T
    stl = _dot_t(hg, whp[...]) + bhp[...]
    e1 = jnp.maximum(_dot_t(ego[...], w1[...]) + b1[...], 0.0)
    e2 = jnp.maximum(_dot_t(e1, w2[...]) + b2[...], 0.0)
    e3 = _dot_t(e2, w3[...]) + b3[...]
    out_o[...] = jnp.concatenate([stl, e3], axis=1)


def _final_call(pooled, counts, whp, bhp, ego, mlp, interpret=False):
    g, h = pooled.shape
    es = ego.shape[1]
    cd = whp.shape[0]
    args = [pooled, counts, whp, bhp, ego]
    for p in mlp:
        args.append(p[0])
        args.append(p[1])
    return pl.pallas_call(
        _final_body,
        out_shape=jax.ShapeDtypeStruct((g, cd + es), F32),
        interpret=interpret,
    )(*args)


def kernel(ego_states, x, edge_index, depths, batch, params):
    n, d = x.shape
    lp0, lp1 = params["layers"]
    h = lp0["Ui"]["w"].shape[0]
    g, es = ego_states.shape
    e = edge_index.shape[1]

    mats = [lp0["Wi"]["w"], lp0["Wo"]["w"], lp0["Wu"]["w"],
            lp1["Wf"]["w"], lp1["Uf"]["w"],
            lp1["Wi"]["w"], lp1["Wo"]["w"], lp1["Wu"]["w"]]
    vecs = [(lp0["Wi"]["b"] + lp0["Ui"]["b"]).reshape(1, h),
            (lp0["Wo"]["b"] + lp0["Uo"]["b"]).reshape(1, h),
            (lp0["Wu"]["b"] + lp0["Uu"]["b"]).reshape(1, h),
            lp1["Wf"]["b"].reshape(1, h),
            lp1["Uf"]["b"].reshape(1, h),
            (lp1["Wi"]["b"] + lp1["Ui"]["b"]).reshape(1, h),
            (lp1["Wo"]["b"] + lp1["Uo"]["b"]).reshape(1, h),
            (lp1["Wu"]["b"] + lp1["Uu"]["b"]).reshape(1, h)]

    bn = 1000
    h1, s_tab, a_tab, xwi, xwo, xwu = _pre_call(x, mats, vecs, bn)

    src = edge_index[0]
    dst = edge_index[1]
    chw = _NC * _NS * _CH
    ep = ((e + chw - 1) // chw) * chw
    np_pad = ((n + 2047) // 2048) * 2048
    if ep != e:
        pad = ep - e
        src = jnp.concatenate([src, jnp.zeros((pad,), jnp.int32)])
        dst = jnp.concatenate([dst, jnp.full((pad,), np_pad, jnp.int32)])

    htp = _make_ht_call(np_pad, h, ep)(src, dst, h1)
    msgp = _make_msg_call(np_pad, h, ep)(src, dst, s_tab, a_tab)

    batch3 = batch.reshape(n // bn, 1, bn)
    pooled, counts = _post_call(
        xwi, xwo, xwu, htp, msgp, batch3,
        [lp1["Ui"]["w"], lp1["Uo"]["w"], lp1["Uu"]["w"]], g, bn)

    mlp = [(p["w"], p["b"].reshape(1, -1)) for p in params["mlp"]]
    return _final_call(pooled, counts,
                       params["hid_proj"]["w"],
                       params["hid_proj"]["b"].reshape(1, h),
                       ego_states, mlp)
